# baseline (device time: 38207 ns/iter reference)
import jax
import jax.numpy as jnp
from jax import lax
from jax.experimental import pallas as pl
from jax.experimental.pallas import tpu as pltpu

C = 8


def kernel(partial, gamma):
    _, m2, d = partial.shape
    m = m2 // 2
    half = m // 2
    r = half // C
    part2d = partial[0]
    gamma2d = gamma.reshape(1, d)

    def body(
        p_ref, g_ref, o_ref,
        ysend_buf, yrecv_buf, xrecv_buf,
        ysend_sems, yrecv_sems, xsend_sems, xrecv_sems,
    ):
        my_x = lax.axis_index("x")
        my_y = lax.axis_index("y")
        my_z = lax.axis_index("z")
        ypeer = (my_x, 1 - my_y, my_z)
        xpeer = (1 - my_x, my_y, my_z)

        barrier = pltpu.get_barrier_semaphore()
        for nbr in (ypeer, xpeer):
            pl.semaphore_signal(
                barrier, inc=1, device_id=nbr,
                device_id_type=pl.DeviceIdType.MESH,
            )
        pl.semaphore_wait(barrier, 2)

        ysend_buf[...] = p_ref[
            pl.ds((1 - my_y) * m + my_x * half, half), :
        ].astype(jnp.bfloat16)

        def y_rdma(k):
            sl = pl.ds(k * r, r)
            return pltpu.make_async_remote_copy(
                src_ref=ysend_buf.at[sl],
                dst_ref=yrecv_buf.at[sl],
                send_sem=ysend_sems.at[k],
                recv_sem=yrecv_sems.at[k],
                device_id=ypeer,
                device_id_type=pl.DeviceIdType.MESH,
            )

        def x_rdma(k):
            sl = pl.ds(k * r, r)
            return pltpu.make_async_remote_copy(
                src_ref=yrecv_buf.at[sl],
                dst_ref=xrecv_buf.at[sl],
                send_sem=xsend_sems.at[k],
                recv_sem=xrecv_sems.at[k],
                device_id=xpeer,
                device_id_type=pl.DeviceIdType.MESH,
            )

        for k in range(C):
            y_rdma(k).start()

        my_base = my_y * m
        direct_off = my_x * half
        fwd_off = (1 - my_x) * half

        def norm_store(local_rows, recv_ref, k):
            sl = pl.ds(k * r, r)
            y = p_ref[pl.ds(my_base + local_rows + k * r, r), :]
            y = y + recv_ref[sl].astype(jnp.float32)
            ms = jnp.mean(y * y, axis=-1, keepdims=True)
            o_ref[pl.ds(local_rows + k * r, r), :] = (
                y * lax.rsqrt(ms + 1e-6) * g_ref[...]
            )

        for k in range(C):
            y_rdma(k).wait_recv()
            x_rdma(k).start()
            norm_store(direct_off, yrecv_buf, k)
            x_rdma(k).wait_recv()
            norm_store(fwd_off, xrecv_buf, k)

        for k in range(C):
            y_rdma(k).wait_send()
            x_rdma(k).wait_send()

    return pl.pallas_call(
        body,
        out_shape=jax.ShapeDtypeStruct((m, d), jnp.float32),
        in_specs=[
            pl.BlockSpec(memory_space=pltpu.VMEM),
            pl.BlockSpec(memory_space=pltpu.VMEM),
        ],
        out_specs=pl.BlockSpec(memory_space=pltpu.VMEM),
        scratch_shapes=[
            pltpu.VMEM((half, d), jnp.bfloat16),
            pltpu.VMEM((half, d), jnp.bfloat16),
            pltpu.VMEM((half, d), jnp.bfloat16),
            pltpu.SemaphoreType.DMA((C,)),
            pltpu.SemaphoreType.DMA((C,)),
            pltpu.SemaphoreType.DMA((C,)),
            pltpu.SemaphoreType.DMA((C,)),
        ],
        compiler_params=pltpu.CompilerParams(collective_id=0),
    )(part2d, gamma2d)


# device time: 26374 ns/iter; 1.4487x vs baseline; 1.4487x over previous
import jax
import jax.numpy as jnp
from jax import lax
from jax.experimental import pallas as pl
from jax.experimental.pallas import tpu as pltpu

C = 4


def kernel(partial, gamma):
    _, m2, d = partial.shape
    m = m2 // 2
    half = m // 2
    r = half // C
    part2d = partial[0]
    gamma2d = gamma.reshape(1, d)

    def body(
        p_ref, g_ref, o_ref,
        ysend_buf, yrecv_buf, xrecv_buf,
        ysend_sems, yrecv_sems, xsend_sems, xrecv_sems,
    ):
        my_x = lax.axis_index("x")
        my_y = lax.axis_index("y")
        my_z = lax.axis_index("z")
        ypeer = (my_x, 1 - my_y, my_z)
        xpeer = (1 - my_x, my_y, my_z)

        barrier = pltpu.get_barrier_semaphore()
        for nbr in (ypeer, xpeer):
            pl.semaphore_signal(
                barrier, inc=1, device_id=nbr,
                device_id_type=pl.DeviceIdType.MESH,
            )
        pl.semaphore_wait(barrier, 2)

        def y_rdma(k):
            sl = pl.ds(k * r, r)
            return pltpu.make_async_remote_copy(
                src_ref=ysend_buf.at[sl],
                dst_ref=yrecv_buf.at[sl],
                send_sem=ysend_sems.at[k],
                recv_sem=yrecv_sems.at[k],
                device_id=ypeer,
                device_id_type=pl.DeviceIdType.MESH,
            )

        def x_rdma(k):
            sl = pl.ds(k * r, r)
            return pltpu.make_async_remote_copy(
                src_ref=yrecv_buf.at[sl],
                dst_ref=xrecv_buf.at[sl],
                send_sem=xsend_sems.at[k],
                recv_sem=xrecv_sems.at[k],
                device_id=xpeer,
                device_id_type=pl.DeviceIdType.MESH,
            )

        peer_base = (1 - my_y) * m + my_x * half
        for k in range(C):
            sl = pl.ds(k * r, r)
            ysend_buf[sl, :] = p_ref[
                pl.ds(peer_base + k * r, r), :
            ].astype(jnp.bfloat16)
            y_rdma(k).start()

        my_base = my_y * m
        direct_off = my_x * half
        fwd_off = (1 - my_x) * half

        def norm_store(local_rows, recv_ref, k):
            sl = pl.ds(k * r, r)
            y = p_ref[pl.ds(my_base + local_rows + k * r, r), :]
            y = y + recv_ref[sl].astype(jnp.float32)
            ms = jnp.mean(y * y, axis=-1, keepdims=True)
            o_ref[pl.ds(local_rows + k * r, r), :] = (
                y * lax.rsqrt(ms + 1e-6) * g_ref[...]
            )

        for k in range(C):
            y_rdma(k).wait_recv()
            x_rdma(k).start()
            norm_store(direct_off, yrecv_buf, k)

        for k in range(C):
            x_rdma(k).wait_recv()
            norm_store(fwd_off, xrecv_buf, k)

        for k in range(C):
            y_rdma(k).wait_send()
            x_rdma(k).wait_send()

    return pl.pallas_call(
        body,
        out_shape=jax.ShapeDtypeStruct((m, d), jnp.float32),
        in_specs=[
            pl.BlockSpec(memory_space=pltpu.VMEM),
            pl.BlockSpec(memory_space=pltpu.VMEM),
        ],
        out_specs=pl.BlockSpec(memory_space=pltpu.VMEM),
        scratch_shapes=[
            pltpu.VMEM((half, d), jnp.bfloat16),
            pltpu.VMEM((half, d), jnp.bfloat16),
            pltpu.VMEM((half, d), jnp.bfloat16),
            pltpu.SemaphoreType.DMA((C,)),
            pltpu.SemaphoreType.DMA((C,)),
            pltpu.SemaphoreType.DMA((C,)),
            pltpu.SemaphoreType.DMA((C,)),
        ],
        compiler_params=pltpu.CompilerParams(collective_id=0),
    )(part2d, gamma2d)


# device time: 26111 ns/iter; 1.4633x vs baseline; 1.0101x over previous
import jax
import jax.numpy as jnp
from jax import lax
from jax.experimental import pallas as pl
from jax.experimental.pallas import tpu as pltpu

SIZES = (16, 32, 48, 80, 112, 112, 112)
OFFS = tuple(sum(SIZES[:i]) for i in range(len(SIZES)))
C = len(SIZES)


def kernel(partial, gamma):
    _, m2, d = partial.shape
    m = m2 // 2
    half = m // 2
    part2d = partial[0]
    gamma2d = gamma.reshape(1, d)

    def body(
        p_ref, g_ref, o_ref,
        ysend_buf, yrecv_buf, xrecv_buf,
        ysend_sems, yrecv_sems, xsend_sems, xrecv_sems,
    ):
        my_x = lax.axis_index("x")
        my_y = lax.axis_index("y")
        my_z = lax.axis_index("z")
        ypeer = (my_x, 1 - my_y, my_z)
        xpeer = (1 - my_x, my_y, my_z)

        barrier = pltpu.get_barrier_semaphore()
        for nbr in (ypeer, xpeer):
            pl.semaphore_signal(
                barrier, inc=1, device_id=nbr,
                device_id_type=pl.DeviceIdType.MESH,
            )

        def y_rdma(k):
            sl = pl.ds(OFFS[k], SIZES[k])
            return pltpu.make_async_remote_copy(
                src_ref=ysend_buf.at[sl],
                dst_ref=yrecv_buf.at[sl],
                send_sem=ysend_sems.at[k],
                recv_sem=yrecv_sems.at[k],
                device_id=ypeer,
                device_id_type=pl.DeviceIdType.MESH,
            )

        def x_rdma(k):
            sl = pl.ds(OFFS[k], SIZES[k])
            return pltpu.make_async_remote_copy(
                src_ref=yrecv_buf.at[sl],
                dst_ref=xrecv_buf.at[sl],
                send_sem=xsend_sems.at[k],
                recv_sem=xrecv_sems.at[k],
                device_id=xpeer,
                device_id_type=pl.DeviceIdType.MESH,
            )

        peer_base = (1 - my_y) * m + my_x * half
        ysend_buf[pl.ds(0, SIZES[0]), :] = p_ref[
            pl.ds(peer_base, SIZES[0]), :
        ].astype(jnp.bfloat16)
        pl.semaphore_wait(barrier, 2)
        y_rdma(0).start()
        for k in range(1, C):
            sl = pl.ds(OFFS[k], SIZES[k])
            ysend_buf[sl, :] = p_ref[
                pl.ds(peer_base + OFFS[k], SIZES[k]), :
            ].astype(jnp.bfloat16)
            y_rdma(k).start()

        my_base = my_y * m
        direct_off = my_x * half
        fwd_off = (1 - my_x) * half

        def norm_store(local_rows, recv_ref, k):
            sl = pl.ds(OFFS[k], SIZES[k])
            y = p_ref[pl.ds(my_base + local_rows + OFFS[k], SIZES[k]), :]
            y = y + recv_ref[sl].astype(jnp.float32)
            ms = jnp.mean(y * y, axis=-1, keepdims=True)
            o_ref[pl.ds(local_rows + OFFS[k], SIZES[k]), :] = (
                y * lax.rsqrt(ms + 1e-6) * g_ref[...]
            )

        for k in range(C):
            y_rdma(k).wait_recv()
            x_rdma(k).start()
            norm_store(direct_off, yrecv_buf, k)

        for k in range(C):
            x_rdma(k).wait_recv()
            norm_store(fwd_off, xrecv_buf, k)

        for k in range(C):
            y_rdma(k).wait_send()
            x_rdma(k).wait_send()

    return pl.pallas_call(
        body,
        out_shape=jax.ShapeDtypeStruct((m, d), jnp.float32),
        in_specs=[
            pl.BlockSpec(memory_space=pltpu.VMEM),
            pl.BlockSpec(memory_space=pltpu.VMEM),
        ],
        out_specs=pl.BlockSpec(memory_space=pltpu.VMEM),
        scratch_shapes=[
            pltpu.VMEM((half, d), jnp.bfloat16),
            pltpu.VMEM((half, d), jnp.bfloat16),
            pltpu.VMEM((half, d), jnp.bfloat16),
            pltpu.SemaphoreType.DMA((C,)),
            pltpu.SemaphoreType.DMA((C,)),
            pltpu.SemaphoreType.DMA((C,)),
            pltpu.SemaphoreType.DMA((C,)),
        ],
        compiler_params=pltpu.CompilerParams(collective_id=0),
    )(part2d, gamma2d)


# device time: 25030 ns/iter; 1.5264x vs baseline; 1.0432x over previous
import jax
import jax.numpy as jnp
from jax import lax
from jax.experimental import pallas as pl
from jax.experimental.pallas import tpu as pltpu

SIZES = (64,) * 8
OFFS = tuple(sum(SIZES[:i]) for i in range(len(SIZES)))
C = len(SIZES)


def kernel(partial, gamma):
    _, m2, d = partial.shape
    m = m2 // 2
    half = m // 2
    part2d = partial[0]
    gamma2d = gamma.reshape(1, d)

    def body(
        p_ref, g_ref, o_ref,
        ysend_buf, yrecv_buf, xrecv_buf,
        ysend_sems, yrecv_sems, xsend_sems, xrecv_sems,
    ):
        my_x = lax.axis_index("x")
        my_y = lax.axis_index("y")
        my_z = lax.axis_index("z")
        ypeer = (my_x, 1 - my_y, my_z)
        xpeer = (1 - my_x, my_y, my_z)

        barrier = pltpu.get_barrier_semaphore()
        for nbr in (ypeer, xpeer):
            pl.semaphore_signal(
                barrier, inc=1, device_id=nbr,
                device_id_type=pl.DeviceIdType.MESH,
            )

        def y_rdma(k):
            sl = pl.ds(OFFS[k], SIZES[k])
            return pltpu.make_async_remote_copy(
                src_ref=ysend_buf.at[sl],
                dst_ref=yrecv_buf.at[sl],
                send_sem=ysend_sems.at[k],
                recv_sem=yrecv_sems.at[k],
                device_id=ypeer,
                device_id_type=pl.DeviceIdType.MESH,
            )

        def x_rdma(k):
            sl = pl.ds(OFFS[k], SIZES[k])
            return pltpu.make_async_remote_copy(
                src_ref=yrecv_buf.at[sl],
                dst_ref=xrecv_buf.at[sl],
                send_sem=xsend_sems.at[k],
                recv_sem=xrecv_sems.at[k],
                device_id=xpeer,
                device_id_type=pl.DeviceIdType.MESH,
            )

        peer_base = (1 - my_y) * m + my_x * half
        ysend_buf[pl.ds(0, SIZES[0]), :] = p_ref[
            pl.ds(peer_base, SIZES[0]), :
        ].astype(jnp.bfloat16)
        pl.semaphore_wait(barrier, 2)
        y_rdma(0).start()
        for k in range(1, C):
            sl = pl.ds(OFFS[k], SIZES[k])
            ysend_buf[sl, :] = p_ref[
                pl.ds(peer_base + OFFS[k], SIZES[k]), :
            ].astype(jnp.bfloat16)
            y_rdma(k).start()

        my_base = my_y * m
        direct_off = my_x * half
        fwd_off = (1 - my_x) * half

        def norm_store(local_rows, recv_ref, k):
            sl = pl.ds(OFFS[k], SIZES[k])
            y = p_ref[pl.ds(my_base + local_rows + OFFS[k], SIZES[k]), :]
            y = y + recv_ref[sl].astype(jnp.float32)
            ms = jnp.mean(y * y, axis=-1, keepdims=True)
            o_ref[pl.ds(local_rows + OFFS[k], SIZES[k]), :] = (
                y * lax.rsqrt(ms + 1e-6) * g_ref[...]
            )

        for k in range(C):
            y_rdma(k).wait_recv()
            x_rdma(k).start()
            norm_store(direct_off, yrecv_buf, k)

        for k in range(C):
            x_rdma(k).wait_recv()
            norm_store(fwd_off, xrecv_buf, k)

        for k in range(C):
            y_rdma(k).wait_send()
            x_rdma(k).wait_send()

    return pl.pallas_call(
        body,
        out_shape=jax.ShapeDtypeStruct((m, d), jnp.float32),
        in_specs=[
            pl.BlockSpec(memory_space=pltpu.VMEM),
            pl.BlockSpec(memory_space=pltpu.VMEM),
        ],
        out_specs=pl.BlockSpec(memory_space=pltpu.VMEM),
        scratch_shapes=[
            pltpu.VMEM((half, d), jnp.bfloat16),
            pltpu.VMEM((half, d), jnp.bfloat16),
            pltpu.VMEM((half, d), jnp.bfloat16),
            pltpu.SemaphoreType.DMA((C,)),
            pltpu.SemaphoreType.DMA((C,)),
            pltpu.SemaphoreType.DMA((C,)),
            pltpu.SemaphoreType.DMA((C,)),
        ],
        compiler_params=pltpu.CompilerParams(collective_id=0),
    )(part2d, gamma2d)


# device time: 24336 ns/iter; 1.5700x vs baseline; 1.0285x over previous
import jax
import jax.numpy as jnp
from jax import lax
from jax.experimental import pallas as pl
from jax.experimental.pallas import tpu as pltpu

R = 64
CD = 8
TAIL = 32
FWD = 512 - TAIL
XSIZES = (R,) * 7 + (TAIL,)
XOFFS = tuple(sum(XSIZES[:i]) for i in range(len(XSIZES)))
CX = len(XSIZES)


def kernel(partial, gamma):
    _, m2, d = partial.shape
    m = m2 // 2
    half = m // 2
    part2d = partial[0]
    gamma2d = gamma.reshape(1, d)

    def body(
        p_ref, g_ref, o_ref,
        ysend_buf, yrecv_buf, xrecv_buf,
        ysend_sems, yrecv_sems, xsend_sems, xrecv_sems,
    ):
        my_x = lax.axis_index("x")
        my_y = lax.axis_index("y")
        my_z = lax.axis_index("z")
        ypeer = (my_x, 1 - my_y, my_z)
        xpeer = (1 - my_x, my_y, my_z)

        barrier = pltpu.get_barrier_semaphore()
        for nbr in (ypeer, xpeer):
            pl.semaphore_signal(
                barrier, inc=1, device_id=nbr,
                device_id_type=pl.DeviceIdType.MESH,
            )

        def y_rdma(k):
            sz = R if k < CD else TAIL
            sl = pl.ds(k * R, sz)
            return pltpu.make_async_remote_copy(
                src_ref=ysend_buf.at[sl],
                dst_ref=yrecv_buf.at[sl],
                send_sem=ysend_sems.at[k],
                recv_sem=yrecv_sems.at[k],
                device_id=ypeer,
                device_id_type=pl.DeviceIdType.MESH,
            )

        def x_rdma(k):
            sl = pl.ds(XOFFS[k], XSIZES[k])
            return pltpu.make_async_remote_copy(
                src_ref=yrecv_buf.at[sl],
                dst_ref=xrecv_buf.at[sl],
                send_sem=xsend_sems.at[k],
                recv_sem=xrecv_sems.at[k],
                device_id=xpeer,
                device_id_type=pl.DeviceIdType.MESH,
            )

        peer_base = (1 - my_y) * m + my_x * half
        peer_tail = (1 - my_y) * m + (1 - my_x) * half + FWD
        ysend_buf[pl.ds(0, R), :] = p_ref[pl.ds(peer_base, R), :].astype(
            jnp.bfloat16
        )
        pl.semaphore_wait(barrier, 2)
        y_rdma(0).start()
        for k in range(1, CD):
            sl = pl.ds(k * R, R)
            ysend_buf[sl, :] = p_ref[
                pl.ds(peer_base + k * R, R), :
            ].astype(jnp.bfloat16)
            y_rdma(k).start()
        ysend_buf[pl.ds(CD * R, TAIL), :] = p_ref[
            pl.ds(peer_tail, TAIL), :
        ].astype(jnp.bfloat16)
        y_rdma(CD).start()

        my_base = my_y * m
        direct_off = my_x * half
        fwd_off = (1 - my_x) * half

        def norm_store(local_rows, recv_ref, off, sz):
            y = p_ref[pl.ds(my_base + local_rows + off, sz), :]
            y = y + recv_ref[pl.ds(off, sz)].astype(jnp.float32)
            ms = jnp.mean(y * y, axis=-1, keepdims=True)
            o_ref[pl.ds(local_rows + off, sz), :] = (
                y * lax.rsqrt(ms + 1e-6) * g_ref[...]
            )

        for k in range(CD):
            y_rdma(k).wait_recv()
            if k < CX:
                x_rdma(k).start()
            norm_store(direct_off, yrecv_buf, k * R, R)
        y_rdma(CD).wait_recv()
        norm_store(fwd_off - CD * R + FWD, yrecv_buf, CD * R, TAIL)

        for k in range(CX):
            x_rdma(k).wait_recv()
            norm_store(fwd_off, xrecv_buf, XOFFS[k], XSIZES[k])

        for k in range(CD + 1):
            y_rdma(k).wait_send()
        for k in range(CX):
            x_rdma(k).wait_send()

    return pl.pallas_call(
        body,
        out_shape=jax.ShapeDtypeStruct((m, d), jnp.float32),
        in_specs=[
            pl.BlockSpec(memory_space=pltpu.VMEM),
            pl.BlockSpec(memory_space=pltpu.VMEM),
        ],
        out_specs=pl.BlockSpec(memory_space=pltpu.VMEM),
        scratch_shapes=[
            pltpu.VMEM((half + TAIL, d), jnp.bfloat16),
            pltpu.VMEM((half + TAIL, d), jnp.bfloat16),
            pltpu.VMEM((half, d), jnp.bfloat16),
            pltpu.SemaphoreType.DMA((CD + 1,)),
            pltpu.SemaphoreType.DMA((CD + 1,)),
            pltpu.SemaphoreType.DMA((CX,)),
            pltpu.SemaphoreType.DMA((CX,)),
        ],
        compiler_params=pltpu.CompilerParams(collective_id=0),
    )(part2d, gamma2d)


# device time: 24334 ns/iter; 1.5701x vs baseline; 1.0001x over previous
import jax
import jax.numpy as jnp
from jax import lax
from jax.experimental import pallas as pl
from jax.experimental.pallas import tpu as pltpu

R = 64
CD = 8
TAIL = 32
FWD = 512 - TAIL
XSIZES = (R,) * 7 + (TAIL,)
XOFFS = tuple(sum(XSIZES[:i]) for i in range(len(XSIZES)))
CX = len(XSIZES)


def kernel(partial, gamma):
    _, m2, d = partial.shape
    m = m2 // 2
    half = m // 2
    part2d = partial[0]
    gamma2d = gamma.reshape(1, d)

    def body(
        p_ref, g_ref, o_ref,
        ysend_buf, yrecv_buf, xrecv_buf,
        ysend_sems, yrecv_sems, xsend_sems, xrecv_sems, xready_sem,
    ):
        my_x = lax.axis_index("x")
        my_y = lax.axis_index("y")
        my_z = lax.axis_index("z")
        ypeer = (my_x, 1 - my_y, my_z)
        xpeer = (1 - my_x, my_y, my_z)

        barrier = pltpu.get_barrier_semaphore()
        pl.semaphore_signal(
            barrier, inc=1, device_id=ypeer,
            device_id_type=pl.DeviceIdType.MESH,
        )
        pl.semaphore_signal(
            xready_sem, inc=1, device_id=xpeer,
            device_id_type=pl.DeviceIdType.MESH,
        )

        def y_rdma(k):
            sz = R if k < CD else TAIL
            sl = pl.ds(k * R, sz)
            return pltpu.make_async_remote_copy(
                src_ref=ysend_buf.at[sl],
                dst_ref=yrecv_buf.at[sl],
                send_sem=ysend_sems.at[k],
                recv_sem=yrecv_sems.at[k],
                device_id=ypeer,
                device_id_type=pl.DeviceIdType.MESH,
            )

        def x_rdma(k):
            sl = pl.ds(XOFFS[k], XSIZES[k])
            return pltpu.make_async_remote_copy(
                src_ref=yrecv_buf.at[sl],
                dst_ref=xrecv_buf.at[sl],
                send_sem=xsend_sems.at[k],
                recv_sem=xrecv_sems.at[k],
                device_id=xpeer,
                device_id_type=pl.DeviceIdType.MESH,
            )

        peer_base = (1 - my_y) * m + my_x * half
        peer_tail = (1 - my_y) * m + (1 - my_x) * half + FWD
        ysend_buf[pl.ds(0, R), :] = p_ref[pl.ds(peer_base, R), :].astype(
            jnp.bfloat16
        )
        pl.semaphore_wait(barrier, 1)
        y_rdma(0).start()
        for k in range(1, CD):
            sl = pl.ds(k * R, R)
            ysend_buf[sl, :] = p_ref[
                pl.ds(peer_base + k * R, R), :
            ].astype(jnp.bfloat16)
            y_rdma(k).start()
        ysend_buf[pl.ds(CD * R, TAIL), :] = p_ref[
            pl.ds(peer_tail, TAIL), :
        ].astype(jnp.bfloat16)
        y_rdma(CD).start()

        my_base = my_y * m
        direct_off = my_x * half
        fwd_off = (1 - my_x) * half

        def norm_store(local_rows, recv_ref, off, sz):
            y = p_ref[pl.ds(my_base + local_rows + off, sz), :]
            y = y + recv_ref[pl.ds(off, sz)].astype(jnp.float32)
            ms = jnp.mean(y * y, axis=-1, keepdims=True)
            o_ref[pl.ds(local_rows + off, sz), :] = (
                y * lax.rsqrt(ms + 1e-6) * g_ref[...]
            )

        for k in range(CD):
            y_rdma(k).wait_recv()
            if k == 0:
                pl.semaphore_wait(xready_sem, 1)
            if k < CX:
                x_rdma(k).start()
            norm_store(direct_off, yrecv_buf, k * R, R)
        y_rdma(CD).wait_recv()
        norm_store(fwd_off - CD * R + FWD, yrecv_buf, CD * R, TAIL)

        for k in range(CX):
            x_rdma(k).wait_recv()
            norm_store(fwd_off, xrecv_buf, XOFFS[k], XSIZES[k])

        for k in range(CD + 1):
            y_rdma(k).wait_send()
        for k in range(CX):
            x_rdma(k).wait_send()

    return pl.pallas_call(
        body,
        out_shape=jax.ShapeDtypeStruct((m, d), jnp.float32),
        in_specs=[
            pl.BlockSpec(memory_space=pltpu.VMEM),
            pl.BlockSpec(memory_space=pltpu.VMEM),
        ],
        out_specs=pl.BlockSpec(memory_space=pltpu.VMEM),
        scratch_shapes=[
            pltpu.VMEM((half + TAIL, d), jnp.bfloat16),
            pltpu.VMEM((half + TAIL, d), jnp.bfloat16),
            pltpu.VMEM((half, d), jnp.bfloat16),
            pltpu.SemaphoreType.DMA((CD + 1,)),
            pltpu.SemaphoreType.DMA((CD + 1,)),
            pltpu.SemaphoreType.DMA((CX,)),
            pltpu.SemaphoreType.DMA((CX,)),
            pltpu.SemaphoreType.REGULAR,
        ],
        compiler_params=pltpu.CompilerParams(collective_id=0),
    )(part2d, gamma2d)
